# baseline (speedup 1.0000x reference)
import functools

import jax
import jax.numpy as jnp
from jax import lax
from jax.experimental import pallas as pl
from jax.experimental.pallas import tpu as pltpu
from jax.experimental.pallas import tpu_sc as plsc

NC = 2
NS = 16
NW = NC * NS


def _pick_cw(n_main):
    best = 128
    for t in range(1, n_main // 128 + 1):
        cw = 128 * t
        if cw > 2048:
            break
        if n_main % cw == 0:
            best = cw
    return best


def _row_minmax_sc(adjacency, n_main):
    n_rows = adjacency.shape[0]
    RW = ((n_rows + NW * 16 - 1) // (NW * 16)) * 16
    NPAD = NW * RW
    GROUPS = RW // 16
    CW = _pick_cw(n_main)
    NCC = n_main // CW
    JMAX = CW // 16

    mesh = plsc.VectorSubcoreMesh(
        core_axis_name="c", subcore_axis_name="s",
        num_cores=NC, num_subcores=NS,
    )

    @functools.partial(
        pl.kernel,
        out_type=[
            jax.ShapeDtypeStruct((NPAD,), jnp.float32),
            jax.ShapeDtypeStruct((NPAD,), jnp.float32),
        ],
        mesh=mesh,
        compiler_params=pltpu.CompilerParams(needs_layout_passes=False),
        scratch_types=[
            pltpu.VMEM((16, CW), jnp.float32),
            pltpu.VMEM((16, CW), jnp.float32),
            pltpu.VMEM((16, 17), jnp.float32),
            pltpu.VMEM((16, 17), jnp.float32),
            pltpu.VMEM((16,), jnp.float32),
            pltpu.VMEM((16,), jnp.float32),
            pltpu.SemaphoreType.DMA,
            pltpu.SemaphoreType.DMA,
        ],
    )
    def rowminmax(adj_hbm, rmax_hbm, rmin_hbm, buf0, buf1, trmax, trmin,
                  stg_max, stg_min, sem0, sem1):
        wid = lax.axis_index("s") * NC + lax.axis_index("c")
        base = wid * RW
        lane = lax.iota(jnp.int32, 16)

        def do_group(g, _):
            rb = base + 16 * g

            @pl.when(rb < n_rows)
            def _():
                bufs = (buf0, buf1)
                sems = (sem0, sem1)
                copies = {}
                for cc in range(min(2, NCC)):
                    copies[cc] = pltpu.async_copy(
                        adj_hbm.at[pl.ds(rb, 16), pl.ds(cc * CW, CW)],
                        bufs[cc % 2], sems[cc % 2])

                for cc in range(NCC):
                    copies[cc].wait()
                    nxt = cc + 2
                    if nxt < NCC:
                        copies[nxt] = pltpu.async_copy(
                            adj_hbm.at[pl.ds(rb, 16), pl.ds(nxt * CW, CW)],
                            bufs[nxt % 2], sems[nxt % 2])
                    buf = bufs[cc % 2]

                    def rstep(r, _, first=(cc == 0)):
                        def jstep(j, acc):
                            am, an = acc
                            v = buf[r, pl.ds(j * 16, 16)]
                            return jnp.maximum(am, v), jnp.minimum(an, v)

                        am0 = jnp.full((16,), -jnp.inf, jnp.float32)
                        an0 = jnp.full((16,), jnp.inf, jnp.float32)
                        am, an = lax.fori_loop(0, JMAX, jstep, (am0, an0))
                        if not first:
                            am = jnp.maximum(am, trmax[r, pl.ds(0, 16)])
                            an = jnp.minimum(an, trmin[r, pl.ds(0, 16)])
                        trmax[r, pl.ds(0, 16)] = am
                        trmin[r, pl.ds(0, 16)] = an
                        return 0

                    lax.fori_loop(0, 16, rstep, 0)

                gmax = jnp.full((16,), -jnp.inf, jnp.float32)
                gmin = jnp.full((16,), jnp.inf, jnp.float32)
                for j in range(16):
                    col = jnp.full((16,), j, jnp.int32)
                    gmax = jnp.maximum(gmax, plsc.load_gather(trmax, [lane, col]))
                    gmin = jnp.minimum(gmin, plsc.load_gather(trmin, [lane, col]))

                stg_max[...] = gmax
                stg_min[...] = gmin
                pltpu.sync_copy(stg_max, rmax_hbm.at[pl.ds(rb, 16)])
                pltpu.sync_copy(stg_min, rmin_hbm.at[pl.ds(rb, 16)])

            return 0

        lax.fori_loop(0, GROUPS, do_group, 0)

    rmax_pad, rmin_pad = rowminmax(adjacency)
    return rmax_pad[:n_rows], rmin_pad[:n_rows]


def _combine_tc(features, W, rmax, rmin, tail):
    m, d = features.shape
    tw = tail.shape[1]
    BM = 1000
    assert m % BM == 0

    def body(f_ref, w_ref, rmx_ref, rmn_ref, tail_ref, o_ref):
        out = jnp.dot(f_ref[...], w_ref[...],
                      preferred_element_type=jnp.float32)
        t = tail_ref[...]
        rmx = jnp.maximum(rmx_ref[...], jnp.max(t, axis=1, keepdims=True))
        rmn = jnp.minimum(rmn_ref[...], jnp.min(t, axis=1, keepdims=True))
        o_ref[...] = jnp.maximum(jnp.maximum(out * rmx, out * rmn), 0.0)

    return pl.pallas_call(
        body,
        grid=(m // BM,),
        in_specs=[
            pl.BlockSpec((BM, d), lambda i: (i, 0)),
            pl.BlockSpec((d, d), lambda i: (0, 0)),
            pl.BlockSpec((BM, 1), lambda i: (i, 0)),
            pl.BlockSpec((BM, 1), lambda i: (i, 0)),
            pl.BlockSpec((BM, tw), lambda i: (i, 0)),
        ],
        out_specs=pl.BlockSpec((BM, d), lambda i: (i, 0)),
        out_shape=jax.ShapeDtypeStruct((m, d), jnp.float32),
    )(features, W, rmax.reshape(m, 1), rmin.reshape(m, 1), tail)


@jax.jit
def kernel(features, adjacency, W):
    n_cols = adjacency.shape[1]
    n_main = (n_cols // 128) * 128
    if n_main == n_cols:
        n_main -= 128
    rmax, rmin = _row_minmax_sc(adjacency, n_main)
    tail = adjacency[:, n_main:]
    return _combine_tc(features, W, rmax, rmin, tail)


# speedup vs baseline: 3.7303x; 3.7303x over previous
import functools

import jax
import jax.numpy as jnp
from jax import lax
from jax.experimental import pallas as pl
from jax.experimental.pallas import tpu as pltpu
from jax.experimental.pallas import tpu_sc as plsc

NC = 2
NS = 16
NW = NC * NS


def _pick_cw(n_main):
    best = 128
    for t in range(1, n_main // 128 + 1):
        cw = 128 * t
        if cw > 3400:
            break
        if n_main % cw == 0:
            best = cw
    return best


def _row_minmax_sc(adjacency, n_main):
    n_rows = adjacency.shape[0]
    RW = ((n_rows + NW * 16 - 1) // (NW * 16)) * 16
    NPAD = NW * RW
    GROUPS = RW // 16
    CW = _pick_cw(n_main)
    NCC = n_main // CW
    JMAX = CW // 16

    mesh = plsc.VectorSubcoreMesh(
        core_axis_name="c", subcore_axis_name="s",
        num_cores=NC, num_subcores=NS,
    )

    @functools.partial(
        pl.kernel,
        out_type=[
            jax.ShapeDtypeStruct((NPAD,), jnp.float32),
            jax.ShapeDtypeStruct((NPAD,), jnp.float32),
        ],
        mesh=mesh,
        compiler_params=pltpu.CompilerParams(needs_layout_passes=False),
        scratch_types=[
            pltpu.VMEM((16, CW), jnp.float32),
            pltpu.VMEM((16, CW), jnp.float32),
            pltpu.VMEM((16, 17), jnp.float32),
            pltpu.VMEM((16, 17), jnp.float32),
            pltpu.VMEM((16,), jnp.float32),
            pltpu.VMEM((16,), jnp.float32),
            pltpu.SemaphoreType.DMA,
            pltpu.SemaphoreType.DMA,
        ],
    )
    def rowminmax(adj_hbm, rmax_hbm, rmin_hbm, buf0, buf1, trmax, trmin,
                  stg_max, stg_min, sem0, sem1):
        wid = lax.axis_index("s") * NC + lax.axis_index("c")
        base = wid * RW
        lane = lax.iota(jnp.int32, 16)

        def do_group(g, _):
            rb = base + 16 * g

            @pl.when(rb < n_rows)
            def _():
                bufs = (buf0, buf1)
                sems = (sem0, sem1)
                copies = {}
                for cc in range(min(2, NCC)):
                    copies[cc] = pltpu.async_copy(
                        adj_hbm.at[pl.ds(rb, 16), pl.ds(cc * CW, CW)],
                        bufs[cc % 2], sems[cc % 2])

                for cc in range(NCC):
                    copies[cc].wait()
                    nxt = cc + 2
                    if nxt < NCC:
                        copies[nxt] = pltpu.async_copy(
                            adj_hbm.at[pl.ds(rb, 16), pl.ds(nxt * CW, CW)],
                            bufs[nxt % 2], sems[nxt % 2])
                    buf = bufs[cc % 2]

                    def rstep(r, _, first=(cc == 0)):
                        def jstep(j, acc):
                            am, an = acc
                            v = buf[r, pl.ds(j * 16, 16)]
                            return jnp.maximum(am, v), jnp.minimum(an, v)

                        am0 = jnp.full((16,), -jnp.inf, jnp.float32)
                        an0 = jnp.full((16,), jnp.inf, jnp.float32)
                        am, an = plsc.parallel_loop(
                            0, JMAX, carry=(am0, an0), unroll=8)(jstep)
                        if not first:
                            am = jnp.maximum(am, trmax[r, pl.ds(0, 16)])
                            an = jnp.minimum(an, trmin[r, pl.ds(0, 16)])
                        trmax[r, pl.ds(0, 16)] = am
                        trmin[r, pl.ds(0, 16)] = an
                        return 0

                    lax.fori_loop(0, 16, rstep, 0)

                gmax = jnp.full((16,), -jnp.inf, jnp.float32)
                gmin = jnp.full((16,), jnp.inf, jnp.float32)
                for j in range(16):
                    col = jnp.full((16,), j, jnp.int32)
                    gmax = jnp.maximum(gmax, plsc.load_gather(trmax, [lane, col]))
                    gmin = jnp.minimum(gmin, plsc.load_gather(trmin, [lane, col]))

                stg_max[...] = gmax
                stg_min[...] = gmin
                pltpu.sync_copy(stg_max, rmax_hbm.at[pl.ds(rb, 16)])
                pltpu.sync_copy(stg_min, rmin_hbm.at[pl.ds(rb, 16)])

            return 0

        lax.fori_loop(0, GROUPS, do_group, 0)

    rmax_pad, rmin_pad = rowminmax(adjacency)
    return rmax_pad[:n_rows], rmin_pad[:n_rows]


def _combine_tc(features, W, rmax, rmin, tail):
    m, d = features.shape
    tw = tail.shape[1]
    BM = 1000
    assert m % BM == 0

    def body(f_ref, w_ref, rmx_ref, rmn_ref, tail_ref, o_ref):
        out = jnp.dot(f_ref[...], w_ref[...],
                      preferred_element_type=jnp.float32)
        t = tail_ref[...]
        rmx = jnp.maximum(rmx_ref[...], jnp.max(t, axis=1, keepdims=True))
        rmn = jnp.minimum(rmn_ref[...], jnp.min(t, axis=1, keepdims=True))
        o_ref[...] = jnp.maximum(jnp.maximum(out * rmx, out * rmn), 0.0)

    return pl.pallas_call(
        body,
        grid=(m // BM,),
        in_specs=[
            pl.BlockSpec((BM, d), lambda i: (i, 0)),
            pl.BlockSpec((d, d), lambda i: (0, 0)),
            pl.BlockSpec((BM, 1), lambda i: (i, 0)),
            pl.BlockSpec((BM, 1), lambda i: (i, 0)),
            pl.BlockSpec((BM, tw), lambda i: (i, 0)),
        ],
        out_specs=pl.BlockSpec((BM, d), lambda i: (i, 0)),
        out_shape=jax.ShapeDtypeStruct((m, d), jnp.float32),
    )(features, W, rmax.reshape(m, 1), rmin.reshape(m, 1), tail)


@jax.jit
def kernel(features, adjacency, W):
    n_cols = adjacency.shape[1]
    n_main = (n_cols // 128) * 128
    if n_main == n_cols:
        n_main -= 128
    rmax, rmin = _row_minmax_sc(adjacency, n_main)
    tail = adjacency[:, n_main:]
    return _combine_tc(features, W, rmax, rmin, tail)


# speedup vs baseline: 5.9025x; 1.5823x over previous
import functools

import jax
import jax.numpy as jnp
from jax import lax
from jax.experimental import pallas as pl
from jax.experimental.pallas import tpu as pltpu
from jax.experimental.pallas import tpu_sc as plsc

NC = 2
NS = 16
NW = NC * NS


def _pick_cw(n_main):
    best = 128
    for t in range(1, n_main // 128 + 1):
        cw = 128 * t
        if cw > 3400:
            break
        if n_main % cw == 0:
            best = cw
    return best


def _row_minmax_sc(adjacency, n_main, row_start):
    n_rows = adjacency.shape[0]
    n_sc = n_rows - row_start
    RW = ((n_sc + NW * 16 - 1) // (NW * 16)) * 16
    NPAD = NW * RW
    GROUPS = RW // 16
    CW = _pick_cw(n_main)
    NCC = n_main // CW
    JMAX = CW // 16

    mesh = plsc.VectorSubcoreMesh(
        core_axis_name="c", subcore_axis_name="s",
        num_cores=NC, num_subcores=NS,
    )

    @functools.partial(
        pl.kernel,
        out_type=[
            jax.ShapeDtypeStruct((NPAD,), jnp.float32),
            jax.ShapeDtypeStruct((NPAD,), jnp.float32),
        ],
        mesh=mesh,
        compiler_params=pltpu.CompilerParams(needs_layout_passes=False),
        scratch_types=[
            pltpu.VMEM((16, CW), jnp.float32),
            pltpu.VMEM((16, CW), jnp.float32),
            pltpu.VMEM((16, 17), jnp.float32),
            pltpu.VMEM((16, 17), jnp.float32),
            pltpu.VMEM((16,), jnp.float32),
            pltpu.VMEM((16,), jnp.float32),
            pltpu.SemaphoreType.DMA,
            pltpu.SemaphoreType.DMA,
        ],
    )
    def rowminmax(adj_hbm, rmax_hbm, rmin_hbm, buf0, buf1, trmax, trmin,
                  stg_max, stg_min, sem0, sem1):
        wid = lax.axis_index("s") * NC + lax.axis_index("c")
        base = row_start + wid * RW
        lane = lax.iota(jnp.int32, 16)

        def do_group(g, _):
            rb = base + 16 * g

            @pl.when(rb < n_rows)
            def _():
                bufs = (buf0, buf1)
                sems = (sem0, sem1)
                copies = {}
                for cc in range(min(2, NCC)):
                    copies[cc] = pltpu.async_copy(
                        adj_hbm.at[pl.ds(rb, 16), pl.ds(cc * CW, CW)],
                        bufs[cc % 2], sems[cc % 2])

                for cc in range(NCC):
                    copies[cc].wait()
                    nxt = cc + 2
                    if nxt < NCC:
                        copies[nxt] = pltpu.async_copy(
                            adj_hbm.at[pl.ds(rb, 16), pl.ds(nxt * CW, CW)],
                            bufs[nxt % 2], sems[nxt % 2])
                    buf = bufs[cc % 2]

                    def rstep(r, _, first=(cc == 0)):
                        def jstep(j, acc):
                            am, an = acc
                            v = buf[r, pl.ds(j * 16, 16)]
                            return jnp.maximum(am, v), jnp.minimum(an, v)

                        am0 = jnp.full((16,), -jnp.inf, jnp.float32)
                        an0 = jnp.full((16,), jnp.inf, jnp.float32)
                        am, an = plsc.parallel_loop(
                            0, JMAX, carry=(am0, an0), unroll=8)(jstep)
                        if not first:
                            am = jnp.maximum(am, trmax[r, pl.ds(0, 16)])
                            an = jnp.minimum(an, trmin[r, pl.ds(0, 16)])
                        trmax[r, pl.ds(0, 16)] = am
                        trmin[r, pl.ds(0, 16)] = an
                        return 0

                    lax.fori_loop(0, 16, rstep, 0)

                gmax = jnp.full((16,), -jnp.inf, jnp.float32)
                gmin = jnp.full((16,), jnp.inf, jnp.float32)
                for j in range(16):
                    col = jnp.full((16,), j, jnp.int32)
                    gmax = jnp.maximum(gmax, plsc.load_gather(trmax, [lane, col]))
                    gmin = jnp.minimum(gmin, plsc.load_gather(trmin, [lane, col]))

                stg_max[...] = gmax
                stg_min[...] = gmin
                pltpu.sync_copy(stg_max, rmax_hbm.at[pl.ds(rb - row_start, 16)])
                pltpu.sync_copy(stg_min, rmin_hbm.at[pl.ds(rb - row_start, 16)])

            return 0

        lax.fori_loop(0, GROUPS, do_group, 0)

    rmax_pad, rmin_pad = rowminmax(adjacency)
    return rmax_pad[:n_sc], rmin_pad[:n_sc]


def _row_minmax_tc(adjacency, n_tc):
    n_cols = adjacency.shape[1]
    bm = 8
    for t in range(1, n_tc // 8 + 1):
        if t * 8 > 512:
            break
        if n_tc % (t * 8) == 0:
            bm = t * 8

    def body(a_ref, mx_ref, mn_ref):
        blk = a_ref[...]
        mx_ref[...] = jnp.max(blk, axis=1, keepdims=True)
        mn_ref[...] = jnp.min(blk, axis=1, keepdims=True)

    return pl.pallas_call(
        body,
        grid=(n_tc // bm,),
        in_specs=[pl.BlockSpec((bm, n_cols), lambda i: (i, 0))],
        out_specs=[
            pl.BlockSpec((bm, 1), lambda i: (i, 0)),
            pl.BlockSpec((bm, 1), lambda i: (i, 0)),
        ],
        out_shape=[
            jax.ShapeDtypeStruct((n_tc, 1), jnp.float32),
            jax.ShapeDtypeStruct((n_tc, 1), jnp.float32),
        ],
    )(adjacency)


def _combine_tc(features, W, rmax, rmin, tail):
    m, d = features.shape
    tw = tail.shape[1]
    BM = 1000
    assert m % BM == 0

    def body(f_ref, w_ref, rmx_ref, rmn_ref, tail_ref, o_ref):
        out = jnp.dot(f_ref[...], w_ref[...],
                      preferred_element_type=jnp.float32)
        t = tail_ref[...]
        rmx = jnp.maximum(rmx_ref[...], jnp.max(t, axis=1, keepdims=True))
        rmn = jnp.minimum(rmn_ref[...], jnp.min(t, axis=1, keepdims=True))
        o_ref[...] = jnp.maximum(jnp.maximum(out * rmx, out * rmn), 0.0)

    return pl.pallas_call(
        body,
        grid=(m // BM,),
        in_specs=[
            pl.BlockSpec((BM, d), lambda i: (i, 0)),
            pl.BlockSpec((d, d), lambda i: (0, 0)),
            pl.BlockSpec((BM, 1), lambda i: (i, 0)),
            pl.BlockSpec((BM, 1), lambda i: (i, 0)),
            pl.BlockSpec((BM, tw), lambda i: (i, 0)),
        ],
        out_specs=pl.BlockSpec((BM, d), lambda i: (i, 0)),
        out_shape=jax.ShapeDtypeStruct((m, d), jnp.float32),
    )(features, W, rmax.reshape(m, 1), rmin.reshape(m, 1), tail)


@jax.jit
def kernel(features, adjacency, W):
    n_rows, n_cols = adjacency.shape
    n_main = (n_cols // 128) * 128
    if n_main == n_cols:
        n_main -= 128
    r_tc = (n_rows * 16 // 26) // 16 * 16
    rmax_sc, rmin_sc = _row_minmax_sc(adjacency, n_main, r_tc)
    rmax_tc, rmin_tc = _row_minmax_tc(adjacency, r_tc)
    rmax = jnp.concatenate([rmax_tc[:, 0], rmax_sc])
    rmin = jnp.concatenate([rmin_tc[:, 0], rmin_sc])
    tail = adjacency[:, n_main:]
    return _combine_tc(features, W, rmax, rmin, tail)


# speedup vs baseline: 6.3427x; 1.0746x over previous
import functools

import jax
import jax.numpy as jnp
from jax import lax
from jax.experimental import pallas as pl
from jax.experimental.pallas import tpu as pltpu
from jax.experimental.pallas import tpu_sc as plsc

NC = 2
NS = 16
NW = NC * NS


def _pick_cw(n_main):
    best = 128
    for t in range(1, n_main // 128 + 1):
        cw = 128 * t
        if cw > 3400:
            break
        if n_main % cw == 0:
            best = cw
    return best


def _row_minmax_sc(adjacency, n_main, row_start):
    n_rows = adjacency.shape[0]
    n_sc = n_rows - row_start
    RW = ((n_sc + NW * 16 - 1) // (NW * 16)) * 16
    NPAD = NW * RW
    GROUPS = RW // 16
    CW = _pick_cw(n_main)
    NCC = n_main // CW
    JMAX = CW // 16

    mesh = plsc.VectorSubcoreMesh(
        core_axis_name="c", subcore_axis_name="s",
        num_cores=NC, num_subcores=NS,
    )

    @functools.partial(
        pl.kernel,
        out_type=[
            jax.ShapeDtypeStruct((NPAD,), jnp.float32),
            jax.ShapeDtypeStruct((NPAD,), jnp.float32),
        ],
        mesh=mesh,
        compiler_params=pltpu.CompilerParams(needs_layout_passes=False),
        scratch_types=[
            pltpu.VMEM((16, CW), jnp.float32),
            pltpu.VMEM((16, CW), jnp.float32),
            pltpu.VMEM((16, 17), jnp.float32),
            pltpu.VMEM((16, 17), jnp.float32),
            pltpu.VMEM((16,), jnp.float32),
            pltpu.VMEM((16,), jnp.float32),
            pltpu.SemaphoreType.DMA,
            pltpu.SemaphoreType.DMA,
        ],
    )
    def rowminmax(adj_hbm, rmax_hbm, rmin_hbm, buf0, buf1, trmax, trmin,
                  stg_max, stg_min, sem0, sem1):
        wid = lax.axis_index("s") * NC + lax.axis_index("c")
        base = row_start + wid * RW
        lane = lax.iota(jnp.int32, 16)

        def do_group(g, _):
            rb = base + 16 * g

            @pl.when(rb < n_rows)
            def _():
                bufs = (buf0, buf1)
                sems = (sem0, sem1)
                copies = {}
                for cc in range(min(2, NCC)):
                    copies[cc] = pltpu.async_copy(
                        adj_hbm.at[pl.ds(rb, 16), pl.ds(cc * CW, CW)],
                        bufs[cc % 2], sems[cc % 2])

                for cc in range(NCC):
                    copies[cc].wait()
                    nxt = cc + 2
                    if nxt < NCC:
                        copies[nxt] = pltpu.async_copy(
                            adj_hbm.at[pl.ds(rb, 16), pl.ds(nxt * CW, CW)],
                            bufs[nxt % 2], sems[nxt % 2])
                    buf = bufs[cc % 2]

                    def rstep(r, _, first=(cc == 0)):
                        def jstep(j, acc):
                            am, an = acc
                            v = buf[r, pl.ds(j * 16, 16)]
                            return jnp.maximum(am, v), jnp.minimum(an, v)

                        am0 = jnp.full((16,), -jnp.inf, jnp.float32)
                        an0 = jnp.full((16,), jnp.inf, jnp.float32)
                        am, an = plsc.parallel_loop(
                            0, JMAX, carry=(am0, an0), unroll=8)(jstep)
                        if not first:
                            am = jnp.maximum(am, trmax[r, pl.ds(0, 16)])
                            an = jnp.minimum(an, trmin[r, pl.ds(0, 16)])
                        trmax[r, pl.ds(0, 16)] = am
                        trmin[r, pl.ds(0, 16)] = an
                        return 0

                    lax.fori_loop(0, 16, rstep, 0)

                gmax = jnp.full((16,), -jnp.inf, jnp.float32)
                gmin = jnp.full((16,), jnp.inf, jnp.float32)
                for j in range(16):
                    col = jnp.full((16,), j, jnp.int32)
                    gmax = jnp.maximum(gmax, plsc.load_gather(trmax, [lane, col]))
                    gmin = jnp.minimum(gmin, plsc.load_gather(trmin, [lane, col]))

                stg_max[...] = gmax
                stg_min[...] = gmin
                pltpu.sync_copy(stg_max, rmax_hbm.at[pl.ds(rb - row_start, 16)])
                pltpu.sync_copy(stg_min, rmin_hbm.at[pl.ds(rb - row_start, 16)])

            return 0

        lax.fori_loop(0, GROUPS, do_group, 0)

    rmax_pad, rmin_pad = rowminmax(adjacency)
    return rmax_pad[:n_sc], rmin_pad[:n_sc]


def _scan_combine_tc(adjacency, features, W, n_tc):
    n_cols = adjacency.shape[1]
    d = features.shape[1]
    bm = 8
    for t in range(1, n_tc // 8 + 1):
        if t * 8 > 256:
            break
        if n_tc % (t * 8) == 0:
            bm = t * 8

    def body(a_ref, f_ref, w_ref, o_ref):
        blk = a_ref[...]
        rmx = jnp.max(blk, axis=1, keepdims=True)
        rmn = jnp.min(blk, axis=1, keepdims=True)
        out = jnp.dot(f_ref[...], w_ref[...],
                      preferred_element_type=jnp.float32)
        o_ref[...] = jnp.maximum(jnp.maximum(out * rmx, out * rmn), 0.0)

    return pl.pallas_call(
        body,
        grid=(n_tc // bm,),
        compiler_params=pltpu.CompilerParams(
            dimension_semantics=("arbitrary",)),
        in_specs=[
            pl.BlockSpec((bm, n_cols), lambda i: (i, 0)),
            pl.BlockSpec((bm, d), lambda i: (i, 0)),
            pl.BlockSpec((d, d), lambda i: (0, 0)),
        ],
        out_specs=pl.BlockSpec((bm, d), lambda i: (i, 0)),
        out_shape=jax.ShapeDtypeStruct((n_tc, d), jnp.float32),
    )(adjacency, features, W)


def _combine_sc_rows(features, W, rmax, rmin, tail, r_tc, bm):
    m, d = features.shape
    n_sc = m - r_tc
    tw = tail.shape[1]
    blk_off = r_tc // bm

    def body(f_ref, w_ref, rmx_ref, rmn_ref, tail_ref, o_ref):
        out = jnp.dot(f_ref[...], w_ref[...],
                      preferred_element_type=jnp.float32)
        t = tail_ref[...]
        rmx = jnp.maximum(rmx_ref[...], jnp.max(t, axis=1, keepdims=True))
        rmn = jnp.minimum(rmn_ref[...], jnp.min(t, axis=1, keepdims=True))
        o_ref[...] = jnp.maximum(jnp.maximum(out * rmx, out * rmn), 0.0)

    return pl.pallas_call(
        body,
        grid=(n_sc // bm,),
        compiler_params=pltpu.CompilerParams(
            dimension_semantics=("arbitrary",)),
        in_specs=[
            pl.BlockSpec((bm, d), lambda i: (i + blk_off, 0)),
            pl.BlockSpec((d, d), lambda i: (0, 0)),
            pl.BlockSpec((bm, 1), lambda i: (i, 0)),
            pl.BlockSpec((bm, 1), lambda i: (i, 0)),
            pl.BlockSpec((bm, tw), lambda i: (i, 0)),
        ],
        out_specs=pl.BlockSpec((bm, d), lambda i: (i, 0)),
        out_shape=jax.ShapeDtypeStruct((n_sc, d), jnp.float32),
    )(features, W, rmax.reshape(n_sc, 1), rmin.reshape(n_sc, 1), tail)


@jax.jit
def kernel(features, adjacency, W):
    n_rows, n_cols = adjacency.shape
    n_main = (n_cols // 128) * 128
    if n_main == n_cols:
        n_main -= 128
    bm2 = 400
    n_sc = (n_rows * 40 // 100) // bm2 * bm2
    r_tc = n_rows - n_sc
    if n_sc == 0 or r_tc % bm2 or n_sc % 16:
        bm2 = 16
        n_sc = (n_rows * 40 // 100) // 16 * 16
        r_tc = n_rows - n_sc
    rmax_sc, rmin_sc = _row_minmax_sc(adjacency, n_main, r_tc)
    agg_tc = _scan_combine_tc(adjacency, features, W, r_tc)
    tail = adjacency[r_tc:, n_main:]
    agg_sc = _combine_sc_rows(features, W, rmax_sc, rmin_sc, tail, r_tc, bm2)
    return jnp.concatenate([agg_tc, agg_sc], axis=0)
